# baseline (speedup 1.0000x reference)
import functools

import jax
import jax.numpy as jnp
from jax.experimental import pallas as pl
from jax.experimental.pallas import tpu as pltpu

D_IN = 4096
H1 = 200
DA = 20
B = 16
L = 1024

JT = 256


def _fused_body(idx_ref, w1t_ref, w4t_ref, at_ref, wsa_ref, bsa_ref,
                w2t_ref, b2_ref, w3t_ref, b3_ref, b4_ref, pc_ref,
                out_ref, hbuf, denom_ref):
    f32 = jnp.float32
    bf16 = jnp.bfloat16
    jt = pl.program_id(0)
    njt = pl.num_programs(0)

    bins = jt * JT + jax.lax.broadcasted_iota(jnp.int32, (1, 1, JT), 2)
    chunk = 128

    def hist_step(c, acc):
        seg = idx_ref[:, pl.ds(c * chunk, chunk)]
        eq = (seg[:, :, None] == bins).astype(f32)
        return acc + jnp.sum(eq, axis=1)

    hj = jax.lax.fori_loop(0, L // chunk, hist_step,
                           jnp.zeros((B, JT), f32))
    hbuf[:, pl.ds(jt * JT, JT)] = hj
    hjb = hj.astype(bf16)

    w1tj = w1t_ref[pl.ds(jt * JT, JT), :]
    m = jax.lax.dot(w1tj, w4t_ref[...], preferred_element_type=f32)
    c = (m * pc_ref[...]).astype(bf16)
    part = jax.lax.dot(hjb, c, preferred_element_type=f32)

    @pl.when(jt == 0)
    def _():
        out_ref[...] = part

    @pl.when(jt > 0)
    def _():
        out_ref[...] += part

    e_jt = jnp.exp(jnp.tanh(
        jax.lax.dot(w1tj, at_ref[...], preferred_element_type=f32)))
    dpart = jax.lax.dot(hjb, e_jt.astype(bf16), preferred_element_type=f32)

    @pl.when(jt == 0)
    def _():
        denom_ref[...] = dpart

    @pl.when(jt > 0)
    def _():
        denom_ref[...] += dpart

    @pl.when(jt == njt - 1)
    def _():
        e_wide = jnp.exp(jnp.tanh(jax.lax.dot_general(
            at_ref[...], w1t_ref[...], (((0,), (1,)), ((), ())),
            preferred_element_type=f32)))
        r = wsa_ref[...] / denom_ref[...]
        w = jax.lax.dot(r.astype(bf16), e_wide.astype(bf16),
                        preferred_element_type=f32)
        f = (hbuf[...] * w).astype(bf16)
        lz = jax.lax.dot(f, w1t_ref[...],
                         preferred_element_type=f32) + bsa_ref[0, 0]
        z = jnp.tanh(lz).astype(bf16)
        z2 = jnp.tanh(jax.lax.dot(z, w2t_ref[...],
                                  preferred_element_type=f32) + b2_ref[...])
        dz = jnp.tanh(jax.lax.dot(z2.astype(bf16), w3t_ref[...],
                                  preferred_element_type=f32) + b3_ref[...])
        pre = jax.lax.dot(dz.astype(bf16), w4t_ref[...],
                          preferred_element_type=f32) + b4_ref[...]
        out_ref[...] = jax.nn.sigmoid(out_ref[...] + pre)


@functools.partial(jax.jit, static_argnames=("interpret",))
def kernel(batch_item_index, place_correlation, W1, W2, b2, W3, b3, W4, b4,
           A, Wsa, bsa, interpret=False):
    f32 = jnp.float32
    bf16 = jnp.bfloat16
    w1t = W1.T.astype(bf16)
    w4t = W4.T.astype(bf16)
    at = A.T.astype(bf16)
    w2t = W2.T.astype(bf16)
    w3t = W3.T.astype(bf16)

    resident = lambda r, c: pl.BlockSpec((r, c), lambda j: (0, 0))
    y = pl.pallas_call(
        _fused_body,
        grid=(D_IN // JT,),
        in_specs=[
            resident(B, L),
            resident(D_IN, H1),
            resident(H1, D_IN),
            resident(H1, DA),
            resident(1, DA),
            resident(1, 1),
            resident(H1, 50),
            resident(1, 50),
            resident(50, H1),
            resident(1, H1),
            resident(1, D_IN),
            pl.BlockSpec((JT, D_IN), lambda j: (j, 0)),
        ],
        out_specs=pl.BlockSpec((B, D_IN), lambda j: (0, 0)),
        out_shape=jax.ShapeDtypeStruct((B, D_IN), f32),
        scratch_shapes=[
            pltpu.VMEM((B, D_IN), f32),
            pltpu.VMEM((B, DA), f32),
        ],
        interpret=interpret,
    )(batch_item_index, w1t, w4t, at, Wsa, bsa.reshape(1, 1),
      w2t, b2.reshape(1, -1), w3t, b3.reshape(1, -1), b4.reshape(1, -1),
      place_correlation)
    return y


# speedup vs baseline: 1.1903x; 1.1903x over previous
import functools

import jax
import jax.numpy as jnp
from jax.experimental import pallas as pl
from jax.experimental.pallas import tpu as pltpu

D_IN = 4096
H1 = 200
DA = 20
B = 16
L = 1024

JT = 1024


def _fused_body(idx_ref, w1t_ref, w4t_ref, at_ref, wsa_ref, bsa_ref,
                w2t_ref, b2_ref, w3t_ref, b3_ref, b4_ref, pc_ref,
                out_ref, hbuf, denom_ref):
    f32 = jnp.float32
    bf16 = jnp.bfloat16
    jt = pl.program_id(0)
    njt = pl.num_programs(0)

    bins = jt * JT + jax.lax.broadcasted_iota(jnp.int32, (1, 1, JT), 2)
    chunk = 128

    def hist_step(c, acc):
        seg = idx_ref[:, pl.ds(c * chunk, chunk)]
        eq = (seg[:, :, None] == bins).astype(f32)
        return acc + jnp.sum(eq, axis=1)

    hj = jax.lax.fori_loop(0, L // chunk, hist_step,
                           jnp.zeros((B, JT), f32))
    hbuf[:, pl.ds(jt * JT, JT)] = hj
    hjb = hj.astype(bf16)

    w1tj = w1t_ref[pl.ds(jt * JT, JT), :]
    m = jax.lax.dot(w1tj, w4t_ref[...], preferred_element_type=f32)
    c = (m * pc_ref[...]).astype(bf16)
    part = jax.lax.dot(hjb, c, preferred_element_type=f32)

    @pl.when(jt == 0)
    def _():
        out_ref[...] = part

    @pl.when(jt > 0)
    def _():
        out_ref[...] += part

    e_jt = jnp.exp(jnp.tanh(
        jax.lax.dot(w1tj, at_ref[...], preferred_element_type=f32)))
    dpart = jax.lax.dot(hjb, e_jt.astype(bf16), preferred_element_type=f32)

    @pl.when(jt == 0)
    def _():
        denom_ref[...] = dpart

    @pl.when(jt > 0)
    def _():
        denom_ref[...] += dpart

    @pl.when(jt == njt - 1)
    def _():
        e_wide = jnp.exp(jnp.tanh(jax.lax.dot_general(
            at_ref[...], w1t_ref[...], (((0,), (1,)), ((), ())),
            preferred_element_type=f32)))
        r = wsa_ref[...] / denom_ref[...]
        w = jax.lax.dot(r.astype(bf16), e_wide.astype(bf16),
                        preferred_element_type=f32)
        f = (hbuf[...] * w).astype(bf16)
        lz = jax.lax.dot(f, w1t_ref[...],
                         preferred_element_type=f32) + bsa_ref[0, 0]
        z = jnp.tanh(lz).astype(bf16)
        z2 = jnp.tanh(jax.lax.dot(z, w2t_ref[...],
                                  preferred_element_type=f32) + b2_ref[...])
        dz = jnp.tanh(jax.lax.dot(z2.astype(bf16), w3t_ref[...],
                                  preferred_element_type=f32) + b3_ref[...])
        pre = jax.lax.dot(dz.astype(bf16), w4t_ref[...],
                          preferred_element_type=f32) + b4_ref[...]
        out_ref[...] = jax.nn.sigmoid(out_ref[...] + pre)


@functools.partial(jax.jit, static_argnames=("interpret",))
def kernel(batch_item_index, place_correlation, W1, W2, b2, W3, b3, W4, b4,
           A, Wsa, bsa, interpret=False):
    f32 = jnp.float32
    bf16 = jnp.bfloat16
    w1t = W1.T.astype(bf16)
    w4t = W4.T.astype(bf16)
    at = A.T.astype(bf16)
    w2t = W2.T.astype(bf16)
    w3t = W3.T.astype(bf16)

    resident = lambda r, c: pl.BlockSpec((r, c), lambda j: (0, 0))
    y = pl.pallas_call(
        _fused_body,
        grid=(D_IN // JT,),
        in_specs=[
            resident(B, L),
            resident(D_IN, H1),
            resident(H1, D_IN),
            resident(H1, DA),
            resident(1, DA),
            resident(1, 1),
            resident(H1, 50),
            resident(1, 50),
            resident(50, H1),
            resident(1, H1),
            resident(1, D_IN),
            pl.BlockSpec((JT, D_IN), lambda j: (j, 0)),
        ],
        out_specs=pl.BlockSpec((B, D_IN), lambda j: (0, 0)),
        out_shape=jax.ShapeDtypeStruct((B, D_IN), f32),
        scratch_shapes=[
            pltpu.VMEM((B, D_IN), f32),
            pltpu.VMEM((B, DA), f32),
        ],
        interpret=interpret,
    )(batch_item_index, w1t, w4t, at, Wsa, bsa.reshape(1, 1),
      w2t, b2.reshape(1, -1), w3t, b3.reshape(1, -1), b4.reshape(1, -1),
      place_correlation)
    return y


# speedup vs baseline: 1.1923x; 1.0016x over previous
import jax
import jax.numpy as jnp
from jax.experimental import pallas as pl
from jax.experimental.pallas import tpu as pltpu

D_IN = 4096
H1 = 200
DA = 20
B = 16
L = 1024

JT = 1024


def _fused_body(idx_ref, w1t_ref, w4t_ref, at_ref, wsa_ref, bsa_ref,
                w2t_ref, b2_ref, w3t_ref, b3_ref, b4_ref, pc_ref,
                out_ref, hbuf, denom_ref):
    f32 = jnp.float32
    bf16 = jnp.bfloat16
    jt = pl.program_id(0)
    njt = pl.num_programs(0)

    bins = jt * JT + jax.lax.broadcasted_iota(jnp.int32, (1, 1, JT), 2)
    chunk = 128

    def hist_step(c, acc):
        seg = idx_ref[:, pl.ds(c * chunk, chunk)]
        eq = (seg[:, :, None] == bins).astype(f32)
        return acc + jnp.sum(eq, axis=1)

    hj = jax.lax.fori_loop(0, L // chunk, hist_step,
                           jnp.zeros((B, JT), f32))
    hbuf[:, pl.ds(jt * JT, JT)] = hj
    hjb = hj.astype(bf16)

    w1tj = w1t_ref[pl.ds(jt * JT, JT), :]
    m = jax.lax.dot(w1tj, w4t_ref[...], preferred_element_type=f32)
    c = (m * pc_ref[...]).astype(bf16)
    part = jax.lax.dot(hjb, c, preferred_element_type=f32)

    @pl.when(jt == 0)
    def _():
        out_ref[...] = part

    @pl.when(jt > 0)
    def _():
        out_ref[...] += part

    e_jt = jnp.exp(jnp.tanh(
        jax.lax.dot(w1tj, at_ref[...], preferred_element_type=f32)))
    dpart = jax.lax.dot(hjb, e_jt.astype(bf16), preferred_element_type=f32)

    @pl.when(jt == 0)
    def _():
        denom_ref[...] = dpart

    @pl.when(jt > 0)
    def _():
        denom_ref[...] += dpart

    @pl.when(jt == njt - 1)
    def _():
        e_wide = jnp.exp(jnp.tanh(jax.lax.dot_general(
            at_ref[...], w1t_ref[...], (((0,), (1,)), ((), ())),
            preferred_element_type=f32)))
        r = wsa_ref[...] / denom_ref[...]
        w = jax.lax.dot(r.astype(bf16), e_wide.astype(bf16),
                        preferred_element_type=f32)
        f = (hbuf[...] * w).astype(bf16)
        lz = jax.lax.dot(f, w1t_ref[...],
                         preferred_element_type=f32) + bsa_ref[0, 0]
        z = jnp.tanh(lz).astype(bf16)
        z2 = jnp.tanh(jax.lax.dot(z, w2t_ref[...],
                                  preferred_element_type=f32) + b2_ref[...])
        dz = jnp.tanh(jax.lax.dot(z2.astype(bf16), w3t_ref[...],
                                  preferred_element_type=f32) + b3_ref[...])
        pre = jax.lax.dot(dz.astype(bf16), w4t_ref[...],
                          preferred_element_type=f32) + b4_ref[...]
        out_ref[...] = jax.nn.sigmoid(out_ref[...] + pre)


@jax.jit
def kernel(batch_item_index, place_correlation, W1, W2, b2, W3, b3, W4, b4,
           A, Wsa, bsa):
    f32 = jnp.float32
    bf16 = jnp.bfloat16
    w1t = W1.T.astype(bf16)
    w4t = W4.T.astype(bf16)
    at = A.T.astype(bf16)
    w2t = W2.T.astype(bf16)
    w3t = W3.T.astype(bf16)

    resident = lambda r, c: pl.BlockSpec((r, c), lambda j: (0, 0))
    y = pl.pallas_call(
        _fused_body,
        grid=(D_IN // JT,),
        in_specs=[
            resident(B, L),
            resident(D_IN, H1),
            resident(H1, D_IN),
            resident(H1, DA),
            resident(1, DA),
            resident(1, 1),
            resident(H1, 50),
            resident(1, 50),
            resident(50, H1),
            resident(1, H1),
            resident(1, D_IN),
            pl.BlockSpec((JT, D_IN), lambda j: (j, 0)),
        ],
        out_specs=pl.BlockSpec((B, D_IN), lambda j: (0, 0)),
        out_shape=jax.ShapeDtypeStruct((B, D_IN), f32),
        scratch_shapes=[
            pltpu.VMEM((B, D_IN), f32),
            pltpu.VMEM((B, DA), f32),
        ],
    )(batch_item_index, w1t, w4t, at, Wsa, bsa.reshape(1, 1),
      w2t, b2.reshape(1, -1), w3t, b3.reshape(1, -1), b4.reshape(1, -1),
      place_correlation)
    return y
